# Initial kernel scaffold; baseline (speedup 1.0000x reference)
import jax
import jax.numpy as jnp
from jax.experimental import pallas as pl


def kernel(x, weight):
    raise NotImplementedError("write your pallas kernel here")


# speedup vs baseline: 9.1924x; 9.1924x over previous
import functools

import jax
import jax.numpy as jnp
from jax.experimental import pallas as pl

_N_WIDTH = 32
_N_ORDER = 2
_N_ELEMENTS = 128
_N_NODES = _N_ELEMENTS * _N_ORDER + 1
_N_SAMPLES = 4096

_BI = 128


def _body(x_ref, w_ref, t_ref, dt_ref, ddt_ref, phi_ref, dphi_ref, ddphi_ref):
    x = x_ref[...]
    xs = x * float(_N_NODES - 1)
    el = jnp.clip(jnp.floor(xs * (1.0 / _N_ORDER)), 0.0, float(_N_ELEMENTS - 1))
    nl = el * float(_N_ORDER)
    t = xs - nl - 1.0

    p0 = 0.5 * t * (t - 1.0)
    p1 = 1.0 - t * t
    p2 = 0.5 * t * (t + 1.0)
    d0 = (t - 0.5) * 256.0
    d1 = t * -512.0
    d2 = (t + 0.5) * 256.0

    nli = nl.astype(jnp.int32)
    rel = jax.lax.broadcasted_iota(jnp.int32, (_BI, _N_NODES), 1) - nli[:, None]
    m0 = rel == 0
    m1 = rel == 1
    m2 = rel == 2

    zero = jnp.zeros((), jnp.float32)
    phi_row = jnp.where(m0, p0[:, None], jnp.where(m1, p1[:, None], jnp.where(m2, p2[:, None], zero)))
    dphi_row = jnp.where(m0, d0[:, None], jnp.where(m1, d1[:, None], jnp.where(m2, d2[:, None], zero)))
    ddphi_row = jnp.where(m0, 65536.0, jnp.where(m1, -131072.0, jnp.where(m2, 65536.0, zero)))

    w = w_ref[...]
    dn = (((1,), (1,)), ((), ()))
    t_ref[...] = jax.lax.dot_general(phi_row, w, dn, preferred_element_type=jnp.float32)
    dt_ref[...] = jax.lax.dot_general(dphi_row, w, dn, preferred_element_type=jnp.float32)
    ddt_ref[...] = jax.lax.dot_general(ddphi_row, w, dn, preferred_element_type=jnp.float32)

    shp = (_BI, _N_WIDTH, _N_NODES)
    phi_ref[...] = jnp.broadcast_to(phi_row[:, None, :], shp)
    dphi_ref[...] = jnp.broadcast_to(dphi_row[:, None, :], shp)
    ddphi_ref[...] = jnp.broadcast_to(ddphi_row[:, None, :], shp)


@jax.jit
def kernel(x, weight):
    grid = (_N_SAMPLES // _BI,)
    out_shapes = (
        jax.ShapeDtypeStruct((_N_SAMPLES, _N_WIDTH), jnp.float32),
        jax.ShapeDtypeStruct((_N_SAMPLES, _N_WIDTH), jnp.float32),
        jax.ShapeDtypeStruct((_N_SAMPLES, _N_WIDTH), jnp.float32),
        jax.ShapeDtypeStruct((_N_SAMPLES, _N_WIDTH, _N_NODES), jnp.float32),
        jax.ShapeDtypeStruct((_N_SAMPLES, _N_WIDTH, _N_NODES), jnp.float32),
        jax.ShapeDtypeStruct((_N_SAMPLES, _N_WIDTH, _N_NODES), jnp.float32),
    )
    out_specs = (
        pl.BlockSpec((_BI, _N_WIDTH), lambda i: (i, 0)),
        pl.BlockSpec((_BI, _N_WIDTH), lambda i: (i, 0)),
        pl.BlockSpec((_BI, _N_WIDTH), lambda i: (i, 0)),
        pl.BlockSpec((_BI, _N_WIDTH, _N_NODES), lambda i: (i, 0, 0)),
        pl.BlockSpec((_BI, _N_WIDTH, _N_NODES), lambda i: (i, 0, 0)),
        pl.BlockSpec((_BI, _N_WIDTH, _N_NODES), lambda i: (i, 0, 0)),
    )
    in_specs = [
        pl.BlockSpec((_BI,), lambda i: (i,)),
        pl.BlockSpec((_N_WIDTH, _N_NODES), lambda i: (0, 0)),
    ]
    return pl.pallas_call(
        _body,
        grid=grid,
        in_specs=in_specs,
        out_specs=out_specs,
        out_shape=out_shapes,
    )(x, weight)
